# baseline (speedup 1.0000x reference)
import jax
import jax.numpy as jnp
from jax import lax
from jax.experimental import pallas as pl
from jax.experimental.pallas import tpu as pltpu
from jax.experimental.pallas import tpu_sc as plsc

BATCH = 16384
EMBED_DIM = 128
NUM_WORKERS = 32
B_PER_W = BATCH // NUM_WORKERS
CHUNK = 128
NCHUNK = B_PER_W // CHUNK
GROUPS = CHUNK // 16


def _body(user_hbm, np_hbm, w_hbm, uid_hbm, nid_hbm, out_hbm,
          u_bufs0, u_bufs1, v_bufs0, v_bufs1, idx_u, idx_v, out_v, w_v,
          sem_u0, sem_u1, sem_v0, sem_v1):
    wid = lax.axis_index("s") * 2 + lax.axis_index("c")

    u_bufs = (u_bufs0, u_bufs1)
    v_bufs = (v_bufs0, v_bufs1)
    sems_u = (sem_u0, sem_u1)
    sems_v = (sem_v0, sem_v1)

    pltpu.sync_copy(uid_hbm.at[pl.ds(wid * NCHUNK, NCHUNK)], idx_u)
    pltpu.sync_copy(nid_hbm.at[pl.ds(wid * NCHUNK, NCHUNK)], idx_v)
    pltpu.sync_copy(w_hbm, w_v)

    iota = lax.iota(jnp.int32, 16)
    zeros16 = jnp.zeros((16,), jnp.int32)
    b_vec = plsc.load_gather(w_v, [zeros16 + 2, zeros16])
    wu = [w_v[0, pl.ds(k * 16, 16)] for k in range(8)]
    wv = [w_v[1, pl.ds(k * 16, 16)] for k in range(8)]

    def fire(c):
        slot = c % 2
        cu = pltpu.async_copy(user_hbm.at[idx_u.at[c]], u_bufs[slot], sems_u[slot])
        cv = pltpu.async_copy(np_hbm.at[idx_v.at[c]], v_bufs[slot], sems_v[slot])
        return cu, cv

    inflight = [fire(0), fire(1)]

    for c in range(NCHUNK):
        slot = c % 2
        cu, cv = inflight[c]
        cu.wait()
        cv.wait()
        u_buf = u_bufs[slot]
        v_buf = v_bufs[slot]

        def gbody(g, _, u_buf=u_buf, v_buf=v_buf, c=c):
            out16 = b_vec
            for r in range(16):
                row = g * 16 + r
                prods = [u_buf[row, pl.ds(k * 16, 16)] * wu[k] for k in range(8)]
                prods += [v_buf[row, pl.ds(k * 16, 16)] * wv[k] for k in range(8)]
                while len(prods) > 1:
                    prods = [prods[i] + prods[i + 1]
                             for i in range(0, len(prods), 2)]
                s = jnp.sum(prods[0])
                out16 = jnp.where(iota == r, s, out16)
            out_v[pl.ds(c * CHUNK + g * 16, 16)] = out16
            return 0

        lax.fori_loop(0, GROUPS, gbody, 0)

        if c + 2 < NCHUNK:
            inflight.append(fire(c + 2))

    pltpu.sync_copy(out_v, out_hbm.at[pl.ds(wid * B_PER_W, B_PER_W)])


@jax.jit
def _two_tower_sc(user_table, nonprofit_table, wflat, uids, nids):
    mesh = plsc.VectorSubcoreMesh(core_axis_name="c", subcore_axis_name="s")
    call = pl.kernel(
        _body,
        mesh=mesh,
        compiler_params=pltpu.CompilerParams(needs_layout_passes=False),
        out_type=jax.ShapeDtypeStruct((BATCH,), jnp.float32),
        scratch_types=[
            pltpu.VMEM((CHUNK, EMBED_DIM), jnp.float32),
            pltpu.VMEM((CHUNK, EMBED_DIM), jnp.float32),
            pltpu.VMEM((CHUNK, EMBED_DIM), jnp.float32),
            pltpu.VMEM((CHUNK, EMBED_DIM), jnp.float32),
            pltpu.VMEM((NCHUNK, CHUNK), jnp.int32),
            pltpu.VMEM((NCHUNK, CHUNK), jnp.int32),
            pltpu.VMEM((B_PER_W,), jnp.float32),
            pltpu.VMEM((3, EMBED_DIM), jnp.float32),
            pltpu.SemaphoreType.DMA,
            pltpu.SemaphoreType.DMA,
            pltpu.SemaphoreType.DMA,
            pltpu.SemaphoreType.DMA,
        ],
    )
    return call(user_table, nonprofit_table, wflat, uids, nids)


def kernel(user_table, nonprofit_table, fc_w, fc_b, user_ids, nonprofit_ids):
    wflat = jnp.concatenate(
        [fc_w.reshape(-1), fc_b.reshape(-1),
         jnp.zeros((EMBED_DIM - 1,), jnp.float32)]
    ).reshape(3, EMBED_DIM)
    uids = user_ids.astype(jnp.int32).reshape(NUM_WORKERS * NCHUNK, CHUNK)
    nids = nonprofit_ids.astype(jnp.int32).reshape(NUM_WORKERS * NCHUNK, CHUNK)
    return _two_tower_sc(user_table, nonprofit_table, wflat, uids, nids)


# speedup vs baseline: 1.2405x; 1.2405x over previous
import jax
import jax.numpy as jnp
from jax import lax
from jax.experimental import pallas as pl
from jax.experimental.pallas import tpu as pltpu
from jax.experimental.pallas import tpu_sc as plsc

BATCH = 16384
EMBED_DIM = 128
NUM_WORKERS = 32
B_PER_W = BATCH // NUM_WORKERS
CHUNK = 128
NCHUNK = B_PER_W // CHUNK
GROUPS = CHUNK // 16


def _body(user_hbm, np_hbm, w_hbm, uid_hbm, nid_hbm, out_hbm,
          u_bufs0, u_bufs1, v_bufs0, v_bufs1, idx_u, idx_v, out_v, w_v,
          sem_u0, sem_u1, sem_v0, sem_v1):
    wid = lax.axis_index("s") * 2 + lax.axis_index("c")

    u_bufs = (u_bufs0, u_bufs1)
    v_bufs = (v_bufs0, v_bufs1)
    sems_u = (sem_u0, sem_u1)
    sems_v = (sem_v0, sem_v1)

    pltpu.sync_copy(uid_hbm.at[pl.ds(wid * NCHUNK, NCHUNK)], idx_u)
    pltpu.sync_copy(nid_hbm.at[pl.ds(wid * NCHUNK, NCHUNK)], idx_v)
    pltpu.sync_copy(w_hbm, w_v)

    iota = lax.iota(jnp.int32, 16)
    zeros16 = jnp.zeros((16,), jnp.int32)
    b_vec = plsc.load_gather(w_v, [zeros16 + 2, zeros16])

    def fire(c):
        slot = c % 2
        cu = pltpu.async_copy(user_hbm.at[idx_u.at[c]], u_bufs[slot], sems_u[slot])
        cv = pltpu.async_copy(np_hbm.at[idx_v.at[c]], v_bufs[slot], sems_v[slot])
        return cu, cv

    inflight = [fire(0), fire(1)]

    for c in range(NCHUNK):
        slot = c % 2
        cu, cv = inflight[c]
        cu.wait()
        cv.wait()
        u_buf = u_bufs[slot]
        v_buf = v_bufs[slot]

        def gbody(g, _, u_buf=u_buf, v_buf=v_buf, c=c):
            accs = [jnp.zeros((16,), jnp.float32) for _ in range(16)]
            for buf, wrow in ((u_buf, 0), (v_buf, 1)):
                for k in range(8):
                    wk = w_v[wrow, pl.ds(k * 16, 16)]
                    for r in range(16):
                        accs[r] = accs[r] + buf[g * 16 + r, pl.ds(k * 16, 16)] * wk
            out16 = b_vec
            for r in range(16):
                out16 = jnp.where(iota == r, jnp.sum(accs[r]), out16)
            out_v[pl.ds(c * CHUNK + g * 16, 16)] = out16
            return 0

        lax.fori_loop(0, GROUPS, gbody, 0)

        if c + 2 < NCHUNK:
            inflight.append(fire(c + 2))

    pltpu.sync_copy(out_v, out_hbm.at[pl.ds(wid * B_PER_W, B_PER_W)])


@jax.jit
def _two_tower_sc(user_table, nonprofit_table, wflat, uids, nids):
    mesh = plsc.VectorSubcoreMesh(core_axis_name="c", subcore_axis_name="s")
    call = pl.kernel(
        _body,
        mesh=mesh,
        compiler_params=pltpu.CompilerParams(needs_layout_passes=False),
        out_type=jax.ShapeDtypeStruct((BATCH,), jnp.float32),
        scratch_types=[
            pltpu.VMEM((CHUNK, EMBED_DIM), jnp.float32),
            pltpu.VMEM((CHUNK, EMBED_DIM), jnp.float32),
            pltpu.VMEM((CHUNK, EMBED_DIM), jnp.float32),
            pltpu.VMEM((CHUNK, EMBED_DIM), jnp.float32),
            pltpu.VMEM((NCHUNK, CHUNK), jnp.int32),
            pltpu.VMEM((NCHUNK, CHUNK), jnp.int32),
            pltpu.VMEM((B_PER_W,), jnp.float32),
            pltpu.VMEM((3, EMBED_DIM), jnp.float32),
            pltpu.SemaphoreType.DMA,
            pltpu.SemaphoreType.DMA,
            pltpu.SemaphoreType.DMA,
            pltpu.SemaphoreType.DMA,
        ],
    )
    return call(user_table, nonprofit_table, wflat, uids, nids)


def kernel(user_table, nonprofit_table, fc_w, fc_b, user_ids, nonprofit_ids):
    wflat = jnp.concatenate(
        [fc_w.reshape(-1), fc_b.reshape(-1),
         jnp.zeros((EMBED_DIM - 1,), jnp.float32)]
    ).reshape(3, EMBED_DIM)
    uids = user_ids.astype(jnp.int32).reshape(NUM_WORKERS * NCHUNK, CHUNK)
    nids = nonprofit_ids.astype(jnp.int32).reshape(NUM_WORKERS * NCHUNK, CHUNK)
    return _two_tower_sc(user_table, nonprofit_table, wflat, uids, nids)
